# baseline (speedup 1.0000x reference)
import functools

import jax
import jax.numpy as jnp
from jax import lax
from jax.experimental import pallas as pl
from jax.experimental.pallas import tpu as pltpu
from jax.experimental.pallas import tpu_sc as plsc

D = 1024
B = 4
T = 2048
NC = 2
NS = 16
NW = NC * NS
PPW = T // NW
C = 16
CPB = PPW // C
NCHUNK = B * CPB
LANES = 16


def _emb_body(idx_hbm, token_hbm, pos_hbm, out_hbm,
              idx_v, pos_v, tok0, tok1, row_ids, sem_p,
              sem_g0, sem_g1, sem_a0, sem_a1, sem_o0, sem_o1):
    wid = lax.axis_index("s") * NC + lax.axis_index("c")
    tok = (tok0, tok1)
    sem_g = (sem_g0, sem_g1)
    sem_a = (sem_a0, sem_a1)
    sem_o = (sem_o0, sem_o1)

    row_ids[...] = lax.iota(jnp.int32, C)
    pltpu.sync_copy(idx_hbm.at[wid], idx_v)
    pos_dma = pltpu.async_copy(pos_hbm.at[pl.ds(wid * PPW, PPW)], pos_v, sem_p)

    def gather(g):
        b, c = divmod(g, CPB)
        p = g & 1
        return pltpu.async_copy(
            token_hbm.at[idx_v.at[b, pl.ds(c * C, C)]], tok[p], sem_g[p])

    gathers = [None] * NCHUNK
    adds = [None] * NCHUNK
    outs = [None] * NCHUNK
    gathers[0] = gather(0)
    for g in range(NCHUNK):
        b, c = divmod(g, CPB)
        p = g & 1
        if g + 1 < NCHUNK:
            if g >= 1 and outs[g - 1] is not None:
                outs[g - 1].wait()
            gathers[g + 1] = gather(g + 1)
        gathers[g].wait()
        if g == 0:
            pos_dma.wait()

        def add_rows(r, carry, _c=c, _p=p):
            t = tok[_p]
            for j in range(D // LANES):
                sl = pl.ds(j * LANES, LANES)
                plsc.addupdate(t.at[r, sl], pos_v[_c * C + r, sl])
            return carry

        lax.fori_loop(0, C, add_rows, 0)
        outs[g] = pltpu.async_copy(
            tok[p], out_hbm.at[b, pl.ds(wid * PPW + c * C, C)], sem_o[p])
    outs[NCHUNK - 2].wait()
    outs[NCHUNK - 1].wait()


_emb_kernel = functools.partial(
    pl.kernel,
    mesh=plsc.VectorSubcoreMesh(core_axis_name="c", subcore_axis_name="s"),
    out_type=jax.ShapeDtypeStruct((B, T, D), jnp.float32),
    scratch_types=[
        pltpu.VMEM((B, PPW), jnp.int32),
        pltpu.VMEM((PPW, D), jnp.float32),
        pltpu.VMEM((C, D), jnp.float32),
        pltpu.VMEM((C, D), jnp.float32),
        pltpu.VMEM((C,), jnp.int32),
        pltpu.SemaphoreType.DMA,
        pltpu.SemaphoreType.DMA,
        pltpu.SemaphoreType.DMA,
        pltpu.SemaphoreType.DMA,
        pltpu.SemaphoreType.DMA,
        pltpu.SemaphoreType.DMA,
        pltpu.SemaphoreType.DMA,
    ],
)(_emb_body)


def kernel(idx, token_emb, pos_emb):
    idx_r = idx.reshape(B, NW, PPW).transpose(1, 0, 2)
    return _emb_kernel(idx_r, token_emb, pos_emb)


# speedup vs baseline: 1.2134x; 1.2134x over previous
import functools

import jax
import jax.numpy as jnp
from jax import lax
from jax.experimental import pallas as pl
from jax.experimental.pallas import tpu as pltpu
from jax.experimental.pallas import tpu_sc as plsc

D = 1024
B = 4
T = 2048
NC = 2
NS = 16
NW = NC * NS
PPW = T // NW
C = 16
CPB = PPW // C
NCHUNK = B * CPB
LANES = 16


def _emb_body(idx_hbm, token_hbm, pos_hbm, out_hbm,
              idx_v, pos_v, tok0, tok1, sem_p,
              sem_g0, sem_g1, sem_o0, sem_o1):
    wid = lax.axis_index("s") * NC + lax.axis_index("c")
    tok = (tok0, tok1)
    sem_g = (sem_g0, sem_g1)
    sem_o = (sem_o0, sem_o1)
    obase = wid * PPW

    pltpu.sync_copy(idx_hbm.at[wid], idx_v)
    pos_dma = pltpu.async_copy(pos_hbm.at[pl.ds(wid * PPW, PPW)], pos_v, sem_p)

    def issue_gather(b, c, p):
        return pltpu.async_copy(
            token_hbm.at[idx_v.at[b, pl.ds(c * C, C)]], tok[p], sem_g[p])

    def wait_gather(p):
        pltpu.make_async_copy(
            token_hbm.at[idx_v.at[0, pl.ds(0, C)]], tok[p], sem_g[p]).wait()

    def issue_out(b, c, p):
        return pltpu.async_copy(
            tok[p], out_hbm.at[b, pl.ds(obase + c * C, C)], sem_o[p])

    def wait_out(p):
        pltpu.make_async_copy(
            tok[p], out_hbm.at[0, pl.ds(0, C)], sem_o[p]).wait()

    def add_chunk(c, p):
        t = tok[p]

        @plsc.parallel_loop(0, C, unroll=1)
        def add_rows(r):
            for j in range(D // LANES):
                sl = pl.ds(j * LANES, LANES)
                plsc.addupdate(t.at[r, sl], pos_v[c * C + r, sl])

    issue_gather(0, 0, 0)
    issue_gather(0, 1, 1)
    wait_gather(0)
    pos_dma.wait()
    add_chunk(0, 0)
    issue_out(0, 0, 0)

    def body(k, carry):
        g1 = 1 + 2 * k
        for dg, p in ((0, 1), (1, 0)):
            g = g1 + dg
            b = g // CPB
            c = g - b * CPB
            wait_out(1 - p)
            gn = g + 1
            bn = gn // CPB
            cn = gn - bn * CPB
            issue_gather(bn, cn, 1 - p)
            wait_gather(p)
            add_chunk(c, p)
            issue_out(b, c, p)
        return carry

    lax.fori_loop(0, (NCHUNK - 2) // 2, body, 0)

    gl = NCHUNK - 1
    bl, cl = divmod(gl, CPB)
    wait_gather(gl & 1)
    add_chunk(cl, gl & 1)
    issue_out(bl, cl, gl & 1)
    wait_out(0)
    wait_out(1)


_emb_kernel = functools.partial(
    pl.kernel,
    mesh=plsc.VectorSubcoreMesh(core_axis_name="c", subcore_axis_name="s"),
    out_type=jax.ShapeDtypeStruct((B, T, D), jnp.float32),
    scratch_types=[
        pltpu.VMEM((B, PPW), jnp.int32),
        pltpu.VMEM((PPW, D), jnp.float32),
        pltpu.VMEM((C, D), jnp.float32),
        pltpu.VMEM((C, D), jnp.float32),
        pltpu.SemaphoreType.DMA,
        pltpu.SemaphoreType.DMA,
        pltpu.SemaphoreType.DMA,
        pltpu.SemaphoreType.DMA,
        pltpu.SemaphoreType.DMA,
    ],
)(_emb_body)


def kernel(idx, token_emb, pos_emb):
    idx_r = idx.reshape(B, NW, PPW).transpose(1, 0, 2)
    return _emb_kernel(idx_r, token_emb, pos_emb)
